# baseline (speedup 1.0000x reference)
import functools

import jax
import jax.numpy as jnp
from jax.experimental import pallas as pl
from jax.experimental.pallas import tpu as pltpu

_LAMBD = 0.5
_BT = 4096


def _partial_kernel(x_any, lab_ref, ce_ref, lab_acc_ref, pred_ref, tp_ref,
                    x_buf, sems):
    j = pl.program_id(0)
    steps = pl.num_programs(0)
    rows = x_any.shape[0] // 2
    xv = x_any.reshape(rows, 256)

    def start(i, slot):
        pltpu.make_async_copy(xv.at[pl.ds(i * _BT, _BT), :],
                              x_buf.at[slot], sems.at[slot]).start()

    @pl.when(j == 0)
    def _():
        start(0, 0)

    @pl.when(j + 1 < steps)
    def _():
        start(j + 1, jax.lax.rem(j + 1, 2))

    slot = jax.lax.rem(j, 2)
    pltpu.make_async_copy(x_buf.at[slot], x_buf.at[slot],
                          sems.at[slot]).wait()

    xb = x_buf[slot]
    d = xb[:, 128:256] - xb[:, 0:128]
    labf = lab_ref[...].astype(jnp.float32)
    v = (1.0 - 2.0 * labf) * d
    ce_t = jnp.maximum(v, 0.0) + jnp.log1p(jnp.exp(-jnp.abs(v)))
    gt = d > 0.0
    predf = jnp.where(gt, 1.0, 0.0)
    tp_t = jnp.where(gt, labf, 0.0)

    def red(t):
        return jnp.sum(t.reshape(_BT // 8, 8, 128), axis=0)

    parts = (red(ce_t), red(labf), red(predf), red(tp_t))
    refs = (ce_ref, lab_acc_ref, pred_ref, tp_ref)

    @pl.when(j == 0)
    def _():
        for r, p in zip(refs, parts):
            r[...] = p

    @pl.when(j > 0)
    def _():
        for r, p in zip(refs, parts):
            r[...] += p


def _combine_kernel(ce_ref, lab_ref, pred_ref, tp_ref, out_ref, *, batch):
    def tot(r):
        return jnp.sum(r[...], axis=(0, 1), keepdims=True)

    ce_sum = tot(ce_ref)
    lab_sum = tot(lab_ref)
    pred_sum = tot(pred_ref)
    tp = tot(tp_ref)

    fn = lab_sum - tp
    fp = pred_sum - tp
    tn = batch - lab_sum - pred_sum + tp

    inv_b = 1.0 / batch
    ce = ce_sum * inv_b
    nonzero = (tp > 0) & (tn > 0) & (fp > 0) & (fn > 0)
    ratio = (tp / jnp.maximum(tp + fn, 1.0)) * (fp / jnp.maximum(fp + tn, 1.0))
    coeff = jnp.where(nonzero,
                      -_LAMBD * jnp.log(jnp.sqrt(jnp.maximum(ratio, 1e-30))),
                      _LAMBD)
    out_ref[...] = ce + coeff * (fn * inv_b)


def kernel(outputs, labels):
    b, c = outputs.shape
    assert c == 2
    rows = b // 128
    steps = rows // _BT

    x2 = outputs.reshape(rows, 128, 2).swapaxes(1, 2).reshape(rows * 2, 128)
    lab2 = labels.astype(jnp.int32).reshape(rows, 128)

    acc = jax.ShapeDtypeStruct((8, 128), jnp.float32)
    parts = pl.pallas_call(
        _partial_kernel,
        grid=(steps,),
        in_specs=[
            pl.BlockSpec(memory_space=pl.ANY),
            pl.BlockSpec((_BT, 128), lambda j: (j, 0)),
        ],
        out_specs=[pl.BlockSpec((8, 128), lambda j: (0, 0))] * 4,
        out_shape=[acc] * 4,
        scratch_shapes=[
            pltpu.VMEM((2, _BT, 256), jnp.float32),
            pltpu.SemaphoreType.DMA((2,)),
        ],
        compiler_params=pltpu.CompilerParams(
            dimension_semantics=("arbitrary",),
        ),
    )(x2, lab2)

    out = pl.pallas_call(
        functools.partial(_combine_kernel, batch=float(b)),
        out_shape=jax.ShapeDtypeStruct((1, 1), jnp.float32),
    )(*parts)
    return out[0, 0]


# speedup vs baseline: 1.0461x; 1.0461x over previous
import functools

import jax
import jax.numpy as jnp
from jax.experimental import pallas as pl
from jax.experimental.pallas import tpu as pltpu

_LAMBD = 0.5
_BT = 8192


def _partial_kernel(x_any, lab_ref, ce_ref, lab_acc_ref, pred_ref, tp_ref,
                    x_buf, sems):
    j = pl.program_id(0)
    steps = pl.num_programs(0)
    rows = x_any.shape[0] // 2
    xv = x_any.reshape(rows, 256)

    def start(i, slot):
        pltpu.make_async_copy(xv.at[pl.ds(i * _BT, _BT), :],
                              x_buf.at[slot], sems.at[slot]).start()

    @pl.when(j == 0)
    def _():
        start(0, 0)

    @pl.when(j + 1 < steps)
    def _():
        start(j + 1, jax.lax.rem(j + 1, 2))

    slot = jax.lax.rem(j, 2)
    pltpu.make_async_copy(x_buf.at[slot], x_buf.at[slot],
                          sems.at[slot]).wait()

    xb = x_buf[slot]
    d = xb[:, 128:256] - xb[:, 0:128]
    labf = lab_ref[...].astype(jnp.float32)
    v = (1.0 - 2.0 * labf) * d
    ce_t = jnp.maximum(v, 0.0) + jnp.log1p(jnp.exp(-jnp.abs(v)))
    gt = d > 0.0
    predf = jnp.where(gt, 1.0, 0.0)
    tp_t = jnp.where(gt, labf, 0.0)

    def red(t):
        return jnp.sum(t.reshape(_BT // 8, 8, 128), axis=0)

    parts = (red(ce_t), red(labf), red(predf), red(tp_t))
    refs = (ce_ref, lab_acc_ref, pred_ref, tp_ref)

    @pl.when(j == 0)
    def _():
        for r, p in zip(refs, parts):
            r[...] = p

    @pl.when(j > 0)
    def _():
        for r, p in zip(refs, parts):
            r[...] += p


def _combine_kernel(ce_ref, lab_ref, pred_ref, tp_ref, out_ref, *, batch):
    def tot(r):
        return jnp.sum(r[...], axis=(0, 1), keepdims=True)

    ce_sum = tot(ce_ref)
    lab_sum = tot(lab_ref)
    pred_sum = tot(pred_ref)
    tp = tot(tp_ref)

    fn = lab_sum - tp
    fp = pred_sum - tp
    tn = batch - lab_sum - pred_sum + tp

    inv_b = 1.0 / batch
    ce = ce_sum * inv_b
    nonzero = (tp > 0) & (tn > 0) & (fp > 0) & (fn > 0)
    ratio = (tp / jnp.maximum(tp + fn, 1.0)) * (fp / jnp.maximum(fp + tn, 1.0))
    coeff = jnp.where(nonzero,
                      -_LAMBD * jnp.log(jnp.sqrt(jnp.maximum(ratio, 1e-30))),
                      _LAMBD)
    out_ref[...] = ce + coeff * (fn * inv_b)


def kernel(outputs, labels):
    b, c = outputs.shape
    assert c == 2
    rows = b // 128
    steps = rows // _BT

    x2 = outputs.reshape(rows, 128, 2).swapaxes(1, 2).reshape(rows * 2, 128)
    lab2 = labels.astype(jnp.int32).reshape(rows, 128)

    acc = jax.ShapeDtypeStruct((8, 128), jnp.float32)
    parts = pl.pallas_call(
        _partial_kernel,
        grid=(steps,),
        in_specs=[
            pl.BlockSpec(memory_space=pl.ANY),
            pl.BlockSpec((_BT, 128), lambda j: (j, 0)),
        ],
        out_specs=[pl.BlockSpec((8, 128), lambda j: (0, 0))] * 4,
        out_shape=[acc] * 4,
        scratch_shapes=[
            pltpu.VMEM((2, _BT, 256), jnp.float32),
            pltpu.SemaphoreType.DMA((2,)),
        ],
        compiler_params=pltpu.CompilerParams(
            dimension_semantics=("arbitrary",),
        ),
    )(x2, lab2)

    out = pl.pallas_call(
        functools.partial(_combine_kernel, batch=float(b)),
        out_shape=jax.ShapeDtypeStruct((1, 1), jnp.float32),
    )(*parts)
    return out[0, 0]
